# baseline (speedup 1.0000x reference)
import functools

import jax
import jax.numpy as jnp
from jax import lax
from jax.experimental import pallas as pl
from jax.experimental.pallas import tpu as pltpu
from jax.experimental.pallas import tpu_sc as plsc

AMP_A = 2.5
AMP_B = 1.5
IOU_THR = 0.3

NC, NS, L = 2, 16, 16
NW = NC * NS


def _make_sc_call(n_pad, m_pad, k_missed):
    per_w = n_pad // NW
    chunks = per_w // L
    mesh = plsc.VectorSubcoreMesh(core_axis_name="c", subcore_axis_name="s",
                                  num_cores=NC, num_subcores=NS)

    def body(assign_hbm, cls_hbm, reg_hbm, ctr_hbm, mi_hbm, mi_next_hbm,
             mt_hbm, out_hbm, mi_v, mi_next_v, mt_v, wtab_v, a_v, c_v, r_v,
             t_v, o_v, dma_sem):
        wid = lax.axis_index("s") * NC + lax.axis_index("c")
        base = wid * per_w
        copies = [
            pltpu.async_copy(mi_hbm, mi_v, dma_sem),
            pltpu.async_copy(mi_next_hbm, mi_next_v, dma_sem),
            pltpu.async_copy(mt_hbm, mt_v, dma_sem),
            pltpu.async_copy(assign_hbm.at[pl.ds(base, per_w)], a_v, dma_sem),
            pltpu.async_copy(cls_hbm.at[pl.ds(base, per_w)], c_v, dma_sem),
            pltpu.async_copy(reg_hbm.at[pl.ds(base, per_w)], r_v, dma_sem),
            pltpu.async_copy(ctr_hbm.at[pl.ds(base, per_w)], t_v, dma_sem),
        ]
        ones = jnp.full((L,), 1.0, jnp.float32)
        for i in range(m_pad // L):
            wtab_v[pl.ds(i * L, L)] = ones
        for c in copies:
            c.wait()
        for k in range(k_missed // L):
            idx = mi_v[pl.ds(k * L, L)]
            nxt = mi_next_v[pl.ds(k * L, L)]
            ty = mt_v[pl.ds(k * L, L)]
            wv = jnp.where(ty == 1, AMP_A, AMP_B).astype(jnp.float32)
            plsc.store_scatter(wtab_v, [idx], wv, mask=idx != nxt)

        unroll = 4

        def step(i, carry):
            ac, ar, at_, np_ = carry
            for u in range(unroll):
                off = (i * unroll + u) * L
                a = a_v[pl.ds(off, L)]
                pos = a >= 0
                safe = jnp.where(pos, a, 0)
                w = plsc.load_gather(wtab_v, [safe])
                wp = jnp.where(pos, w, 0.0)
                ac = ac + c_v[pl.ds(off, L)] * wp
                ar = ar + r_v[pl.ds(off, L)] * wp
                at_ = at_ + t_v[pl.ds(off, L)] * wp
                np_ = np_ + jnp.where(pos, 1.0, 0.0)
            return ac, ar, at_, np_

        z = jnp.zeros((L,), jnp.float32)
        ac, ar, at_, np_ = lax.fori_loop(0, chunks // unroll, step,
                                         (z, z, z, z))
        o_v[pl.ds(0, L)] = ac
        o_v[pl.ds(L, L)] = ar
        o_v[pl.ds(2 * L, L)] = at_
        o_v[pl.ds(3 * L, L)] = np_
        pltpu.sync_copy(o_v, out_hbm.at[wid])

    return pl.kernel(
        body,
        out_type=jax.ShapeDtypeStruct((NW, 4 * L), jnp.float32),
        mesh=mesh,
        scratch_types=[
            pltpu.VMEM((k_missed,), jnp.int32),
            pltpu.VMEM((k_missed,), jnp.int32),
            pltpu.VMEM((k_missed,), jnp.int32),
            pltpu.VMEM((m_pad,), jnp.float32),
            pltpu.VMEM((per_w,), jnp.int32),
            pltpu.VMEM((per_w,), jnp.float32),
            pltpu.VMEM((per_w,), jnp.float32),
            pltpu.VMEM((per_w,), jnp.float32),
            pltpu.VMEM((4 * L,), jnp.float32),
            pltpu.SemaphoreType.DMA,
        ],
        compiler_params=pltpu.CompilerParams(needs_layout_passes=False),
    )


def _make_tc_call(n_pad, m, k_missed, blk):
    grid_n = n_pad // blk

    kt_n = k_missed // 16
    pt_n = blk // 128
    grp = 8
    bf = jnp.bfloat16

    def body(pbT, clsT, aT, gt, mi_col, out, mbb_ref, acc_ref):
        pid = pl.program_id(0)
        sc13 = jnp.float32(1.0 + IOU_THR)

        @pl.when(pid == 0)
        def _init():
            k_iota = lax.broadcasted_iota(jnp.int32, (k_missed, m), 1)
            oh = (mi_col[...] == k_iota).astype(jnp.float32)
            mb = jnp.dot(oh, gt[...], preferred_element_type=jnp.float32)
            mbb_ref[0] = jnp.broadcast_to(
                (sc13 * mb[:, 0:1]).astype(bf), (k_missed, 128))
            mbb_ref[1] = jnp.broadcast_to(
                (sc13 * mb[:, 2:3]).astype(bf), (k_missed, 128))
            mbb_ref[2] = jnp.broadcast_to(
                mb[:, 1:2].astype(bf), (k_missed, 128))
            mbb_ref[3] = jnp.broadcast_to(
                mb[:, 3:4].astype(bf), (k_missed, 128))
            bam = IOU_THR * ((mb[:, 2:3] - mb[:, 0:1]) *
                             (mb[:, 3:4] - mb[:, 1:2]))
            mbb_ref[4] = jnp.broadcast_to(bam.astype(bf), (k_missed, 128))
            acc_ref[...] = jnp.zeros((1, blk), jnp.float32)

        psx1 = (sc13 * pbT[0:1, :]).astype(bf)
        psx2 = (sc13 * pbT[2:3, :]).astype(bf)
        py1 = pbT[1:2, :].astype(bf)
        py2 = pbT[3:4, :].astype(bf)
        bap = IOU_THR * ((pbT[2:3, :] - pbT[0:1, :]) *
                         (pbT[3:4, :] - pbT[1:2, :]))

        for pg in range(pt_n // grp):
            sls = [slice((pg * grp + j) * 128, (pg * grp + j + 1) * 128)
                   for j in range(grp)]
            a1 = [jnp.broadcast_to(psx1[:, s], (16, 128)) for s in sls]
            a2 = [jnp.broadcast_to(psx2[:, s], (16, 128)) for s in sls]
            b1 = [jnp.broadcast_to(py1[:, s], (16, 128)) for s in sls]
            b2 = [jnp.broadcast_to(py2[:, s], (16, 128)) for s in sls]
            sm = [None] * grp
            for kt in range(kt_n):
                ks = slice(kt * 16, (kt + 1) * 16)
                m1 = mbb_ref[0, ks, :]
                m2 = mbb_ref[1, ks, :]
                n1 = mbb_ref[2, ks, :]
                n2 = mbb_ref[3, ks, :]
                bm = mbb_ref[4, ks, :]
                for j in range(grp):
                    ix = jnp.maximum(
                        jnp.minimum(a2[j], m2) - jnp.maximum(a1[j], m1),
                        bf(0))
                    iy = jnp.maximum(
                        jnp.minimum(b2[j], n2) - jnp.maximum(b1[j], n1),
                        bf(0))
                    sc = ix * iy - bm
                    sm[j] = sc if kt == 0 else jnp.maximum(sm[j], sc)
            for j, s in enumerate(sls):
                smf = sm[j].astype(jnp.float32)
                nearmax = jnp.max(smf, axis=0, keepdims=True)
                keep = (aT[:, s] < 0) & (nearmax < bap[:, s])
                acc_ref[0:1, s] += jnp.where(keep, clsT[:, s], 0.0)

        @pl.when(pid == grid_n - 1)
        def _fin():
            out[...] = jnp.sum(acc_ref[...], axis=(0, 1), keepdims=True)

    return pl.pallas_call(
        body,
        grid=(grid_n,),
        in_specs=[
            pl.BlockSpec((4, blk), lambda i: (0, i)),
            pl.BlockSpec((1, blk), lambda i: (0, i)),
            pl.BlockSpec((1, blk), lambda i: (0, i)),
            pl.BlockSpec((m, 4), lambda i: (0, 0)),
            pl.BlockSpec((k_missed, 1), lambda i: (0, 0)),
        ],
        out_specs=pl.BlockSpec((1, 1), lambda i: (0, 0)),
        out_shape=jax.ShapeDtypeStruct((1, 1), jnp.float32),
        scratch_shapes=[pltpu.VMEM((5, k_missed, 128), jnp.bfloat16),
                        pltpu.VMEM((1, blk), jnp.float32)],
        compiler_params=pltpu.CompilerParams(
            dimension_semantics=("arbitrary",)),
    )


def kernel(cls_losses, reg_losses, ctr_losses, point_boxes, gt_boxes,
           point_gt_indices, missed_indices, missed_is_type_a):
    n = cls_losses.shape[0]
    m = gt_boxes.shape[0]
    k_missed = missed_indices.shape[0]
    n_pad = ((n + 8 * NW * L - 1) // (8 * NW * L)) * (8 * NW * L)
    m_pad = ((m + L - 1) // L) * L
    pad = n_pad - n

    a_pad = jnp.pad(point_gt_indices, (0, pad), constant_values=-1)
    cls_pad = jnp.pad(cls_losses, (0, pad))
    reg_pad = jnp.pad(reg_losses, (0, pad))
    ctr_pad = jnp.pad(ctr_losses, (0, pad))
    pb_pad = jnp.pad(point_boxes, ((0, pad), (0, 0)))
    mi_next = jnp.concatenate(
        [missed_indices[1:], jnp.full((1,), -1, jnp.int32)])

    s2 = _make_tc_call(n_pad, m, k_missed, 2048)(
        pb_pad.T, cls_pad[None, :], a_pad[None, :], gt_boxes,
        missed_indices[:, None])
    sc_out = _make_sc_call(n_pad, m_pad, k_missed)(
        a_pad, cls_pad, reg_pad, ctr_pad, missed_indices, mi_next,
        missed_is_type_a)

    parts = sc_out.reshape(NW, 4, L).sum(axis=(0, 2))
    npos = jnp.maximum(parts[3], 1.0)
    cls_loss = (parts[0] + s2[0, 0]) / npos
    reg_loss = parts[1] / npos
    ctr_loss = parts[2] / npos
    return cls_loss, reg_loss, ctr_loss


# speedup vs baseline: 1.0353x; 1.0353x over previous
import functools

import jax
import jax.numpy as jnp
from jax import lax
from jax.experimental import pallas as pl
from jax.experimental.pallas import tpu as pltpu
from jax.experimental.pallas import tpu_sc as plsc

AMP_A = 2.5
AMP_B = 1.5
IOU_THR = 0.3

NC, NS, L = 1, 16, 16
NW = NC * NS


def _make_sc_call(n_pad, m_pad, k_missed):
    per_w = n_pad // NW
    chunks = per_w // L
    mesh = plsc.VectorSubcoreMesh(core_axis_name="c", subcore_axis_name="s",
                                  num_cores=NC, num_subcores=NS)

    def body(assign_hbm, cls_hbm, reg_hbm, ctr_hbm, mi_hbm, mi_next_hbm,
             mt_hbm, out_hbm, mi_v, mi_next_v, mt_v, wtab_v, a_v, c_v, r_v,
             t_v, o_v, dma_sem):
        wid = lax.axis_index("s") * NC + lax.axis_index("c")
        base = wid * per_w
        copies = [
            pltpu.async_copy(mi_hbm, mi_v, dma_sem),
            pltpu.async_copy(mi_next_hbm, mi_next_v, dma_sem),
            pltpu.async_copy(mt_hbm, mt_v, dma_sem),
            pltpu.async_copy(assign_hbm.at[pl.ds(base, per_w)], a_v, dma_sem),
            pltpu.async_copy(cls_hbm.at[pl.ds(base, per_w)], c_v, dma_sem),
            pltpu.async_copy(reg_hbm.at[pl.ds(base, per_w)], r_v, dma_sem),
            pltpu.async_copy(ctr_hbm.at[pl.ds(base, per_w)], t_v, dma_sem),
        ]
        ones = jnp.full((L,), 1.0, jnp.float32)
        for i in range(m_pad // L):
            wtab_v[pl.ds(i * L, L)] = ones
        for c in copies:
            c.wait()
        for k in range(k_missed // L):
            idx = mi_v[pl.ds(k * L, L)]
            nxt = mi_next_v[pl.ds(k * L, L)]
            ty = mt_v[pl.ds(k * L, L)]
            wv = jnp.where(ty == 1, AMP_A, AMP_B).astype(jnp.float32)
            plsc.store_scatter(wtab_v, [idx], wv, mask=idx != nxt)

        unroll = 4

        def step(i, carry):
            ac, ar, at_, np_ = carry
            for u in range(unroll):
                off = (i * unroll + u) * L
                a = a_v[pl.ds(off, L)]
                pos = a >= 0
                safe = jnp.where(pos, a, 0)
                w = plsc.load_gather(wtab_v, [safe])
                wp = jnp.where(pos, w, 0.0)
                ac = ac + c_v[pl.ds(off, L)] * wp
                ar = ar + r_v[pl.ds(off, L)] * wp
                at_ = at_ + t_v[pl.ds(off, L)] * wp
                np_ = np_ + jnp.where(pos, 1.0, 0.0)
            return ac, ar, at_, np_

        z = jnp.zeros((L,), jnp.float32)
        ac, ar, at_, np_ = lax.fori_loop(0, chunks // unroll, step,
                                         (z, z, z, z))
        o_v[pl.ds(0, L)] = ac
        o_v[pl.ds(L, L)] = ar
        o_v[pl.ds(2 * L, L)] = at_
        o_v[pl.ds(3 * L, L)] = np_
        pltpu.sync_copy(o_v, out_hbm.at[wid])

    return pl.kernel(
        body,
        out_type=jax.ShapeDtypeStruct((NW, 4 * L), jnp.float32),
        mesh=mesh,
        scratch_types=[
            pltpu.VMEM((k_missed,), jnp.int32),
            pltpu.VMEM((k_missed,), jnp.int32),
            pltpu.VMEM((k_missed,), jnp.int32),
            pltpu.VMEM((m_pad,), jnp.float32),
            pltpu.VMEM((per_w,), jnp.int32),
            pltpu.VMEM((per_w,), jnp.float32),
            pltpu.VMEM((per_w,), jnp.float32),
            pltpu.VMEM((per_w,), jnp.float32),
            pltpu.VMEM((4 * L,), jnp.float32),
            pltpu.SemaphoreType.DMA,
        ],
        compiler_params=pltpu.CompilerParams(needs_layout_passes=False),
    )


def _make_tc_call(n_pad, m, k_missed, blk):
    grid_n = n_pad // blk

    kt_n = k_missed // 16
    pt_n = blk // 128
    grp = 8
    bf = jnp.bfloat16

    def body(pbT, clsT, aT, gt, mi_col, out, mbb_ref, acc_ref):
        pid = pl.program_id(0)
        sc13 = jnp.float32(1.0 + IOU_THR)

        @pl.when(pid == 0)
        def _init():
            k_iota = lax.broadcasted_iota(jnp.int32, (k_missed, m), 1)
            oh = (mi_col[...] == k_iota).astype(jnp.float32)
            mb = jnp.dot(oh, gt[...], preferred_element_type=jnp.float32)
            mbb_ref[0] = jnp.broadcast_to(
                (sc13 * mb[:, 0:1]).astype(bf), (k_missed, 128))
            mbb_ref[1] = jnp.broadcast_to(
                (sc13 * mb[:, 2:3]).astype(bf), (k_missed, 128))
            mbb_ref[2] = jnp.broadcast_to(
                mb[:, 1:2].astype(bf), (k_missed, 128))
            mbb_ref[3] = jnp.broadcast_to(
                mb[:, 3:4].astype(bf), (k_missed, 128))
            bam = IOU_THR * ((mb[:, 2:3] - mb[:, 0:1]) *
                             (mb[:, 3:4] - mb[:, 1:2]))
            mbb_ref[4] = jnp.broadcast_to(bam.astype(bf), (k_missed, 128))
            acc_ref[...] = jnp.zeros((1, blk), jnp.float32)

        psx1 = (sc13 * pbT[0:1, :]).astype(bf)
        psx2 = (sc13 * pbT[2:3, :]).astype(bf)
        py1 = pbT[1:2, :].astype(bf)
        py2 = pbT[3:4, :].astype(bf)
        bap = IOU_THR * ((pbT[2:3, :] - pbT[0:1, :]) *
                         (pbT[3:4, :] - pbT[1:2, :]))

        for pg in range(pt_n // grp):
            sls = [slice((pg * grp + j) * 128, (pg * grp + j + 1) * 128)
                   for j in range(grp)]
            a1 = [jnp.broadcast_to(psx1[:, s], (16, 128)) for s in sls]
            a2 = [jnp.broadcast_to(psx2[:, s], (16, 128)) for s in sls]
            b1 = [jnp.broadcast_to(py1[:, s], (16, 128)) for s in sls]
            b2 = [jnp.broadcast_to(py2[:, s], (16, 128)) for s in sls]
            sm = [None] * grp
            for kt in range(kt_n):
                ks = slice(kt * 16, (kt + 1) * 16)
                m1 = mbb_ref[0, ks, :]
                m2 = mbb_ref[1, ks, :]
                n1 = mbb_ref[2, ks, :]
                n2 = mbb_ref[3, ks, :]
                bm = mbb_ref[4, ks, :]
                for j in range(grp):
                    ix = jnp.maximum(
                        jnp.minimum(a2[j], m2) - jnp.maximum(a1[j], m1),
                        bf(0))
                    iy = jnp.maximum(
                        jnp.minimum(b2[j], n2) - jnp.maximum(b1[j], n1),
                        bf(0))
                    sc = ix * iy - bm
                    sm[j] = sc if kt == 0 else jnp.maximum(sm[j], sc)
            for j, s in enumerate(sls):
                smf = sm[j].astype(jnp.float32)
                nearmax = jnp.max(smf, axis=0, keepdims=True)
                keep = (aT[:, s] < 0) & (nearmax < bap[:, s])
                acc_ref[0:1, s] += jnp.where(keep, clsT[:, s], 0.0)

        @pl.when(pid == grid_n - 1)
        def _fin():
            out[...] = jnp.sum(acc_ref[...], axis=(0, 1), keepdims=True)

    return pl.pallas_call(
        body,
        grid=(grid_n,),
        in_specs=[
            pl.BlockSpec((4, blk), lambda i: (0, i)),
            pl.BlockSpec((1, blk), lambda i: (0, i)),
            pl.BlockSpec((1, blk), lambda i: (0, i)),
            pl.BlockSpec((m, 4), lambda i: (0, 0)),
            pl.BlockSpec((k_missed, 1), lambda i: (0, 0)),
        ],
        out_specs=pl.BlockSpec((1, 1), lambda i: (0, 0)),
        out_shape=jax.ShapeDtypeStruct((1, 1), jnp.float32),
        scratch_shapes=[pltpu.VMEM((5, k_missed, 128), jnp.bfloat16),
                        pltpu.VMEM((1, blk), jnp.float32)],
        compiler_params=pltpu.CompilerParams(
            dimension_semantics=("arbitrary",)),
    )


def kernel(cls_losses, reg_losses, ctr_losses, point_boxes, gt_boxes,
           point_gt_indices, missed_indices, missed_is_type_a):
    n = cls_losses.shape[0]
    m = gt_boxes.shape[0]
    k_missed = missed_indices.shape[0]
    n_pad = ((n + 8 * NW * L - 1) // (8 * NW * L)) * (8 * NW * L)
    m_pad = ((m + L - 1) // L) * L
    pad = n_pad - n

    a_pad = jnp.pad(point_gt_indices, (0, pad), constant_values=-1)
    cls_pad = jnp.pad(cls_losses, (0, pad))
    reg_pad = jnp.pad(reg_losses, (0, pad))
    ctr_pad = jnp.pad(ctr_losses, (0, pad))
    pb_pad = jnp.pad(point_boxes, ((0, pad), (0, 0)))
    mi_next = jnp.concatenate(
        [missed_indices[1:], jnp.full((1,), -1, jnp.int32)])

    s2 = _make_tc_call(n_pad, m, k_missed, 2048)(
        pb_pad.T, cls_pad[None, :], a_pad[None, :], gt_boxes,
        missed_indices[:, None])
    sc_out = _make_sc_call(n_pad, m_pad, k_missed)(
        a_pad, cls_pad, reg_pad, ctr_pad, missed_indices, mi_next,
        missed_is_type_a)

    parts = sc_out.reshape(NW, 4, L).sum(axis=(0, 2))
    npos = jnp.maximum(parts[3], 1.0)
    cls_loss = (parts[0] + s2[0, 0]) / npos
    reg_loss = parts[1] / npos
    ctr_loss = parts[2] / npos
    return cls_loss, reg_loss, ctr_loss


# speedup vs baseline: 1.0388x; 1.0034x over previous
import functools

import jax
import jax.numpy as jnp
from jax import lax
from jax.experimental import pallas as pl
from jax.experimental.pallas import tpu as pltpu
from jax.experimental.pallas import tpu_sc as plsc

AMP_A = 2.5
AMP_B = 1.5
IOU_THR = 0.3

NC, NS, L = 1, 16, 16
NW = NC * NS


def _make_sc_call(n_pad, m_pad, k_missed):
    per_w = n_pad // NW
    chunks = per_w // L
    mesh = plsc.VectorSubcoreMesh(core_axis_name="c", subcore_axis_name="s",
                                  num_cores=NC, num_subcores=NS)

    def body(assign_hbm, cls_hbm, reg_hbm, ctr_hbm, mi_hbm, mi_next_hbm,
             mt_hbm, out_hbm, mi_v, mi_next_v, mt_v, wtab_v, a_v, c_v, r_v,
             t_v, o_v, dma_sem):
        wid = lax.axis_index("s") * NC + lax.axis_index("c")
        base = wid * per_w
        copies = [
            pltpu.async_copy(mi_hbm, mi_v, dma_sem),
            pltpu.async_copy(mi_next_hbm, mi_next_v, dma_sem),
            pltpu.async_copy(mt_hbm, mt_v, dma_sem),
            pltpu.async_copy(assign_hbm.at[pl.ds(base, per_w)], a_v, dma_sem),
            pltpu.async_copy(cls_hbm.at[pl.ds(base, per_w)], c_v, dma_sem),
            pltpu.async_copy(reg_hbm.at[pl.ds(base, per_w)], r_v, dma_sem),
            pltpu.async_copy(ctr_hbm.at[pl.ds(base, per_w)], t_v, dma_sem),
        ]
        ones = jnp.full((L,), 1.0, jnp.float32)
        for i in range(m_pad // L):
            wtab_v[pl.ds(i * L, L)] = ones
        for c in copies:
            c.wait()
        for k in range(k_missed // L):
            idx = mi_v[pl.ds(k * L, L)]
            nxt = mi_next_v[pl.ds(k * L, L)]
            ty = mt_v[pl.ds(k * L, L)]
            wv = jnp.where(ty == 1, AMP_A, AMP_B).astype(jnp.float32)
            plsc.store_scatter(wtab_v, [idx], wv, mask=idx != nxt)

        unroll = 4

        def step(i, carry):
            ac, ar, at_, np_ = carry
            for u in range(unroll):
                off = (i * unroll + u) * L
                a = a_v[pl.ds(off, L)]
                pos = a >= 0
                safe = jnp.where(pos, a, 0)
                w = plsc.load_gather(wtab_v, [safe])
                wp = jnp.where(pos, w, 0.0)
                ac = ac + c_v[pl.ds(off, L)] * wp
                ar = ar + r_v[pl.ds(off, L)] * wp
                at_ = at_ + t_v[pl.ds(off, L)] * wp
                np_ = np_ + jnp.where(pos, 1.0, 0.0)
            return ac, ar, at_, np_

        z = jnp.zeros((L,), jnp.float32)
        ac, ar, at_, np_ = lax.fori_loop(0, chunks // unroll, step,
                                         (z, z, z, z))
        o_v[pl.ds(0, L)] = ac
        o_v[pl.ds(L, L)] = ar
        o_v[pl.ds(2 * L, L)] = at_
        o_v[pl.ds(3 * L, L)] = np_
        pltpu.sync_copy(o_v, out_hbm.at[wid])

    return pl.kernel(
        body,
        out_type=jax.ShapeDtypeStruct((NW, 4 * L), jnp.float32),
        mesh=mesh,
        scratch_types=[
            pltpu.VMEM((k_missed,), jnp.int32),
            pltpu.VMEM((k_missed,), jnp.int32),
            pltpu.VMEM((k_missed,), jnp.int32),
            pltpu.VMEM((m_pad,), jnp.float32),
            pltpu.VMEM((per_w,), jnp.int32),
            pltpu.VMEM((per_w,), jnp.float32),
            pltpu.VMEM((per_w,), jnp.float32),
            pltpu.VMEM((per_w,), jnp.float32),
            pltpu.VMEM((4 * L,), jnp.float32),
            pltpu.SemaphoreType.DMA,
        ],
        compiler_params=pltpu.CompilerParams(needs_layout_passes=False,
                                             skip_device_barrier=True),
    )


def _make_tc_call(n_pad, m, k_missed, blk):
    grid_n = n_pad // blk

    kt_n = k_missed // 16
    pt_n = blk // 128
    grp = 8
    bf = jnp.bfloat16

    def body(pbT, clsT, aT, gt, mi_col, out, mbb_ref, acc_ref):
        pid = pl.program_id(0)
        sc13 = jnp.float32(1.0 + IOU_THR)

        @pl.when(pid == 0)
        def _init():
            k_iota = lax.broadcasted_iota(jnp.int32, (k_missed, m), 1)
            oh = (mi_col[...] == k_iota).astype(jnp.float32)
            mb = jnp.dot(oh, gt[...], preferred_element_type=jnp.float32)
            mbb_ref[0] = jnp.broadcast_to(
                (sc13 * mb[:, 0:1]).astype(bf), (k_missed, 128))
            mbb_ref[1] = jnp.broadcast_to(
                (sc13 * mb[:, 2:3]).astype(bf), (k_missed, 128))
            mbb_ref[2] = jnp.broadcast_to(
                mb[:, 1:2].astype(bf), (k_missed, 128))
            mbb_ref[3] = jnp.broadcast_to(
                mb[:, 3:4].astype(bf), (k_missed, 128))
            bam = IOU_THR * ((mb[:, 2:3] - mb[:, 0:1]) *
                             (mb[:, 3:4] - mb[:, 1:2]))
            mbb_ref[4] = jnp.broadcast_to(bam.astype(bf), (k_missed, 128))
            acc_ref[...] = jnp.zeros((1, blk), jnp.float32)

        psx1 = (sc13 * pbT[0:1, :]).astype(bf)
        psx2 = (sc13 * pbT[2:3, :]).astype(bf)
        py1 = pbT[1:2, :].astype(bf)
        py2 = pbT[3:4, :].astype(bf)
        bap = IOU_THR * ((pbT[2:3, :] - pbT[0:1, :]) *
                         (pbT[3:4, :] - pbT[1:2, :]))

        for pg in range(pt_n // grp):
            sls = [slice((pg * grp + j) * 128, (pg * grp + j + 1) * 128)
                   for j in range(grp)]
            a1 = [jnp.broadcast_to(psx1[:, s], (16, 128)) for s in sls]
            a2 = [jnp.broadcast_to(psx2[:, s], (16, 128)) for s in sls]
            b1 = [jnp.broadcast_to(py1[:, s], (16, 128)) for s in sls]
            b2 = [jnp.broadcast_to(py2[:, s], (16, 128)) for s in sls]
            sm = [None] * grp
            for kt in range(kt_n):
                ks = slice(kt * 16, (kt + 1) * 16)
                m1 = mbb_ref[0, ks, :]
                m2 = mbb_ref[1, ks, :]
                n1 = mbb_ref[2, ks, :]
                n2 = mbb_ref[3, ks, :]
                bm = mbb_ref[4, ks, :]
                for j in range(grp):
                    ix = jnp.maximum(
                        jnp.minimum(a2[j], m2) - jnp.maximum(a1[j], m1),
                        bf(0))
                    iy = jnp.maximum(
                        jnp.minimum(b2[j], n2) - jnp.maximum(b1[j], n1),
                        bf(0))
                    sc = ix * iy - bm
                    sm[j] = sc if kt == 0 else jnp.maximum(sm[j], sc)
            for j, s in enumerate(sls):
                smf = sm[j].astype(jnp.float32)
                nearmax = jnp.max(smf, axis=0, keepdims=True)
                keep = (aT[:, s] < 0) & (nearmax < bap[:, s])
                acc_ref[0:1, s] += jnp.where(keep, clsT[:, s], 0.0)

        @pl.when(pid == grid_n - 1)
        def _fin():
            out[...] = jnp.sum(acc_ref[...], axis=(0, 1), keepdims=True)

    return pl.pallas_call(
        body,
        grid=(grid_n,),
        in_specs=[
            pl.BlockSpec((4, blk), lambda i: (0, i)),
            pl.BlockSpec((1, blk), lambda i: (0, i)),
            pl.BlockSpec((1, blk), lambda i: (0, i)),
            pl.BlockSpec((m, 4), lambda i: (0, 0)),
            pl.BlockSpec((k_missed, 1), lambda i: (0, 0)),
        ],
        out_specs=pl.BlockSpec((1, 1), lambda i: (0, 0)),
        out_shape=jax.ShapeDtypeStruct((1, 1), jnp.float32),
        scratch_shapes=[pltpu.VMEM((5, k_missed, 128), jnp.bfloat16),
                        pltpu.VMEM((1, blk), jnp.float32)],
        compiler_params=pltpu.CompilerParams(
            dimension_semantics=("arbitrary",),
            skip_device_barrier=True),
    )


def kernel(cls_losses, reg_losses, ctr_losses, point_boxes, gt_boxes,
           point_gt_indices, missed_indices, missed_is_type_a):
    n = cls_losses.shape[0]
    m = gt_boxes.shape[0]
    k_missed = missed_indices.shape[0]
    n_pad = ((n + 8 * NW * L - 1) // (8 * NW * L)) * (8 * NW * L)
    m_pad = ((m + L - 1) // L) * L
    pad = n_pad - n

    a_pad = jnp.pad(point_gt_indices, (0, pad), constant_values=-1)
    cls_pad = jnp.pad(cls_losses, (0, pad))
    reg_pad = jnp.pad(reg_losses, (0, pad))
    ctr_pad = jnp.pad(ctr_losses, (0, pad))
    pb_pad = jnp.pad(point_boxes, ((0, pad), (0, 0)))
    mi_next = jnp.concatenate(
        [missed_indices[1:], jnp.full((1,), -1, jnp.int32)])

    s2 = _make_tc_call(n_pad, m, k_missed, 2048)(
        pb_pad.T, cls_pad[None, :], a_pad[None, :], gt_boxes,
        missed_indices[:, None])
    sc_out = _make_sc_call(n_pad, m_pad, k_missed)(
        a_pad, cls_pad, reg_pad, ctr_pad, missed_indices, mi_next,
        missed_is_type_a)

    parts = sc_out.reshape(NW, 4, L).sum(axis=(0, 2))
    npos = jnp.maximum(parts[3], 1.0)
    cls_loss = (parts[0] + s2[0, 0]) / npos
    reg_loss = parts[1] / npos
    ctr_loss = parts[2] / npos
    return cls_loss, reg_loss, ctr_loss


# speedup vs baseline: 1.2146x; 1.1692x over previous
import functools

import jax
import jax.numpy as jnp
from jax import lax
from jax.experimental import pallas as pl
from jax.experimental.pallas import tpu as pltpu
from jax.experimental.pallas import tpu_sc as plsc

AMP_A = 2.5
AMP_B = 1.5
IOU_THR = 0.3

NC, NS, L = 1, 16, 16
NW = NC * NS


def _make_sc_call(n_pad, m_pad, k_missed):
    per_w = n_pad // NW
    chunks = per_w // L
    mesh = plsc.VectorSubcoreMesh(core_axis_name="c", subcore_axis_name="s",
                                  num_cores=NC, num_subcores=NS)

    def body(assign_hbm, cls_hbm, reg_hbm, ctr_hbm, mi_hbm, mi_next_hbm,
             mt_hbm, out_hbm, mi_v, mi_next_v, mt_v, wtab_v, a_v, c_v, r_v,
             t_v, o_v, dma_sem):
        wid = lax.axis_index("s") * NC + lax.axis_index("c")
        base = wid * per_w
        copies = [
            pltpu.async_copy(mi_hbm, mi_v, dma_sem),
            pltpu.async_copy(mi_next_hbm, mi_next_v, dma_sem),
            pltpu.async_copy(mt_hbm, mt_v, dma_sem),
            pltpu.async_copy(assign_hbm.at[pl.ds(base, per_w)], a_v, dma_sem),
            pltpu.async_copy(cls_hbm.at[pl.ds(base, per_w)], c_v, dma_sem),
            pltpu.async_copy(reg_hbm.at[pl.ds(base, per_w)], r_v, dma_sem),
            pltpu.async_copy(ctr_hbm.at[pl.ds(base, per_w)], t_v, dma_sem),
        ]
        ones = jnp.full((L,), 1.0, jnp.float32)
        for i in range(m_pad // L):
            wtab_v[pl.ds(i * L, L)] = ones
        for c in copies:
            c.wait()
        for k in range(k_missed // L):
            idx = mi_v[pl.ds(k * L, L)]
            nxt = mi_next_v[pl.ds(k * L, L)]
            ty = mt_v[pl.ds(k * L, L)]
            wv = jnp.where(ty == 1, AMP_A, AMP_B).astype(jnp.float32)
            plsc.store_scatter(wtab_v, [idx], wv, mask=idx != nxt)

        unroll = 4

        def step(i, carry):
            ac, ar, at_, np_ = carry
            for u in range(unroll):
                off = (i * unroll + u) * L
                a = a_v[pl.ds(off, L)]
                pos = a >= 0
                safe = jnp.where(pos, a, 0)
                w = plsc.load_gather(wtab_v, [safe])
                wp = jnp.where(pos, w, 0.0)
                ac = ac + c_v[pl.ds(off, L)] * wp
                ar = ar + r_v[pl.ds(off, L)] * wp
                at_ = at_ + t_v[pl.ds(off, L)] * wp
                np_ = np_ + jnp.where(pos, 1.0, 0.0)
            return ac, ar, at_, np_

        z = jnp.zeros((L,), jnp.float32)
        ac, ar, at_, np_ = lax.fori_loop(0, chunks // unroll, step,
                                         (z, z, z, z))
        o_v[pl.ds(0, L)] = ac
        o_v[pl.ds(L, L)] = ar
        o_v[pl.ds(2 * L, L)] = at_
        o_v[pl.ds(3 * L, L)] = np_
        pltpu.sync_copy(o_v, out_hbm.at[wid])

    return pl.kernel(
        body,
        out_type=jax.ShapeDtypeStruct((NW, 4 * L), jnp.float32),
        mesh=mesh,
        scratch_types=[
            pltpu.VMEM((k_missed,), jnp.int32),
            pltpu.VMEM((k_missed,), jnp.int32),
            pltpu.VMEM((k_missed,), jnp.int32),
            pltpu.VMEM((m_pad,), jnp.float32),
            pltpu.VMEM((per_w,), jnp.int32),
            pltpu.VMEM((per_w,), jnp.float32),
            pltpu.VMEM((per_w,), jnp.float32),
            pltpu.VMEM((per_w,), jnp.float32),
            pltpu.VMEM((4 * L,), jnp.float32),
            pltpu.SemaphoreType.DMA,
        ],
        compiler_params=pltpu.CompilerParams(needs_layout_passes=False,
                                             skip_device_barrier=True),
    )


def _make_tc_call(n_pad, m, k_missed, blk):
    grid_n = n_pad // blk

    kt_n = k_missed // 16
    pt_n = blk // 128
    grp = 8
    bf = jnp.bfloat16

    def body(pbT, clsT, aT, gt, mi_col, out, mbb_ref, acc_ref):
        pid = pl.program_id(0)
        sc13 = jnp.float32(1.0 + IOU_THR)

        @pl.when(pid == 0)
        def _init():
            k_iota = lax.broadcasted_iota(jnp.int32, (k_missed, m), 1)
            oh = (mi_col[...] == k_iota).astype(jnp.float32)
            mb = jnp.dot(oh, gt[...], preferred_element_type=jnp.float32)
            mbb_ref[0] = jnp.broadcast_to(
                (sc13 * mb[:, 0:1]).astype(bf), (k_missed, 128))
            mbb_ref[1] = jnp.broadcast_to(
                (sc13 * mb[:, 2:3]).astype(bf), (k_missed, 128))
            mbb_ref[2] = jnp.broadcast_to(
                mb[:, 1:2].astype(bf), (k_missed, 128))
            mbb_ref[3] = jnp.broadcast_to(
                mb[:, 3:4].astype(bf), (k_missed, 128))
            bam = IOU_THR * ((mb[:, 2:3] - mb[:, 0:1]) *
                             (mb[:, 3:4] - mb[:, 1:2]))
            mbb_ref[4] = jnp.broadcast_to(bam.astype(bf), (k_missed, 128))
            acc_ref[...] = jnp.zeros((1, blk), jnp.float32)

        psx1 = (sc13 * pbT[0:1, :]).astype(bf)
        psx2 = (sc13 * pbT[2:3, :]).astype(bf)
        py1 = pbT[1:2, :].astype(bf)
        py2 = pbT[3:4, :].astype(bf)
        bap = IOU_THR * ((pbT[2:3, :] - pbT[0:1, :]) *
                         (pbT[3:4, :] - pbT[1:2, :]))

        for pg in range(pt_n // grp):
            sls = [slice((pg * grp + j) * 128, (pg * grp + j + 1) * 128)
                   for j in range(grp)]
            a1 = [jnp.broadcast_to(psx1[:, s], (16, 128)) for s in sls]
            a2 = [jnp.broadcast_to(psx2[:, s], (16, 128)) for s in sls]
            b1 = [jnp.broadcast_to(py1[:, s], (16, 128)) for s in sls]
            b2 = [jnp.broadcast_to(py2[:, s], (16, 128)) for s in sls]
            sm = [None] * grp
            for kt in range(kt_n):
                ks = slice(kt * 16, (kt + 1) * 16)
                m1 = mbb_ref[0, ks, :]
                m2 = mbb_ref[1, ks, :]
                n1 = mbb_ref[2, ks, :]
                n2 = mbb_ref[3, ks, :]
                bm = mbb_ref[4, ks, :]
                for j in range(grp):
                    ix = jnp.maximum(
                        jnp.minimum(a2[j], m2) - jnp.maximum(a1[j], m1),
                        bf(0))
                    iy = jnp.maximum(
                        jnp.minimum(b2[j], n2) - jnp.maximum(b1[j], n1),
                        bf(0))
                    sc = ix * iy - bm
                    sm[j] = sc if kt == 0 else jnp.maximum(sm[j], sc)
            for j, s in enumerate(sls):
                smf = sm[j].astype(jnp.float32)
                nearmax = jnp.max(smf, axis=0, keepdims=True)
                keep = (aT[:, s] < 0) & (nearmax < bap[:, s])
                acc_ref[0:1, s] += jnp.where(keep, clsT[:, s], 0.0)

        @pl.when(pid == grid_n - 1)
        def _fin():
            out[...] = jnp.sum(acc_ref[...], axis=(0, 1), keepdims=True)

    return pl.pallas_call(
        body,
        grid=(grid_n,),
        in_specs=[
            pl.BlockSpec((4, blk), lambda i: (0, i)),
            pl.BlockSpec((1, blk), lambda i: (0, i)),
            pl.BlockSpec((1, blk), lambda i: (0, i)),
            pl.BlockSpec((m, 4), lambda i: (0, 0)),
            pl.BlockSpec((k_missed, 1), lambda i: (0, 0)),
        ],
        out_specs=pl.BlockSpec((1, 1), lambda i: (0, 0)),
        out_shape=jax.ShapeDtypeStruct((1, 1), jnp.float32),
        scratch_shapes=[pltpu.VMEM((5, k_missed, 128), jnp.bfloat16),
                        pltpu.VMEM((1, blk), jnp.float32)],
        compiler_params=pltpu.CompilerParams(
            dimension_semantics=("arbitrary",),
            skip_device_barrier=True),
    )


def kernel(cls_losses, reg_losses, ctr_losses, point_boxes, gt_boxes,
           point_gt_indices, missed_indices, missed_is_type_a):
    n = cls_losses.shape[0]
    m = gt_boxes.shape[0]
    k_missed = missed_indices.shape[0]
    n_pad = ((n + 8 * NW * L - 1) // (8 * NW * L)) * (8 * NW * L)
    m_pad = ((m + L - 1) // L) * L
    pad = n_pad - n

    a_pad = jnp.pad(point_gt_indices, (0, pad), constant_values=-1)
    cls_pad = jnp.pad(cls_losses, (0, pad))
    reg_pad = jnp.pad(reg_losses, (0, pad))
    ctr_pad = jnp.pad(ctr_losses, (0, pad))
    pb_pad = jnp.pad(point_boxes, ((0, pad), (0, 0)))
    mi_next = jnp.concatenate(
        [missed_indices[1:], jnp.full((1,), -1, jnp.int32)])

    s2 = _make_tc_call(n_pad, m, k_missed, 4096)(
        pb_pad.T, cls_pad[None, :], a_pad[None, :], gt_boxes,
        missed_indices[:, None])
    sc_out = _make_sc_call(n_pad, m_pad, k_missed)(
        a_pad, cls_pad, reg_pad, ctr_pad, missed_indices, mi_next,
        missed_is_type_a)

    parts = sc_out.reshape(NW, 4, L).sum(axis=(0, 2))
    npos = jnp.maximum(parts[3], 1.0)
    cls_loss = (parts[0] + s2[0, 0]) / npos
    reg_loss = parts[1] / npos
    ctr_loss = parts[2] / npos
    return cls_loss, reg_loss, ctr_loss
